# baseline (speedup 1.0000x reference)
import math

import jax
import jax.numpy as jnp
from jax.experimental import pallas as pl
from jax.experimental.pallas import tpu as pltpu

C_IN = 64
NSAMPLE = 16
VOXEL = 0.25

_QB = 256


def _knn_body(qmat_ref, bmat_ref, out_ref):
    q = qmat_ref[0]
    bm = bmat_ref[0]
    s = jnp.dot(q, bm, preferred_element_type=jnp.float32)
    qq = (q[:, 0:1] * q[:, 0:1] + q[:, 1:2] * q[:, 1:2]) + q[:, 2:3] * q[:, 2:3]
    bb = bm[3:4, :]
    d = (qq - 2.0 * s) + bb
    iota = jax.lax.broadcasted_iota(jnp.int32, d.shape, 1)
    big = jnp.int32(2 ** 30)
    for r in range(NSAMPLE):
        m = jnp.min(d, axis=1, keepdims=True)
        sel = jnp.min(jnp.where(d <= m, iota, big), axis=1, keepdims=True)
        out_ref[0, :, r:r + 1] = sel
        d = jnp.where(iota == sel, jnp.float32(jnp.inf), d)


def _knn_pallas(x1, x2):
    B, N, _ = x1.shape
    q_all = jnp.concatenate([x1, x2], axis=0)
    b_all = jnp.concatenate([x2, x1], axis=0)
    zeros1 = jnp.zeros((2 * B, N, 1), jnp.float32)
    zeros4 = jnp.zeros((2 * B, N, 4), jnp.float32)
    qmat = jnp.concatenate([q_all, zeros1, zeros4], axis=-1)
    bb = jnp.sum(b_all * b_all, axis=-1, keepdims=True)
    bmat_rows = jnp.concatenate([b_all, bb, zeros4], axis=-1)
    bmat = bmat_rows.transpose(0, 2, 1)

    grid = (2 * B, N // _QB)
    out = pl.pallas_call(
        _knn_body,
        grid=grid,
        in_specs=[
            pl.BlockSpec((1, _QB, 8), lambda g, i: (g, i, 0)),
            pl.BlockSpec((1, 8, N), lambda g, i: (g, 0, 0)),
        ],
        out_specs=pl.BlockSpec((1, _QB, NSAMPLE), lambda g, i: (g, i, 0)),
        out_shape=jax.ShapeDtypeStruct((2 * B, N, NSAMPLE), jnp.int32),
    )(qmat, bmat)
    return out[:B], out[B:]


def _dense(x, l):
    return x @ l["W"].T + l["b"]


def _bn(x, p, eps=1e-05):
    m = jnp.mean(x, axis=(0, 1), keepdims=True)
    v = jnp.var(x, axis=(0, 1), keepdims=True)
    return (x - m) / jnp.sqrt(v + eps) * p["g"] + p["b"]


def _leaky(x):
    return jnp.where(x > 0, x, 0.1 * x)


def _gather(pts, idx):
    return jax.vmap(lambda p, i: p[i])(pts, idx)


def _posenc(pe_raw, pec):
    scale = 2.0 * math.pi
    e = pe_raw / (1.0 + 1e-06) * scale
    i = jnp.arange(8, dtype=jnp.float32)
    dim_t = 10000.0 ** (2.0 * jnp.floor(i / 2.0) / 8.0)
    outs = []
    for c in range(3):
        p = e[:, c:c + 1] / dim_t
        p = jnp.stack([jnp.sin(p[:, 0::2]), jnp.cos(p[:, 1::2])], axis=2).reshape(p.shape[0], -1)
        outs.append(p)
    pos = jnp.concatenate(outs, axis=1)
    return _dense(pos, pec)


def _intra_patch(pos_diff, wp):
    Bq, Nq, S, _ = pos_diff.shape
    pd = pos_diff.reshape(-1, S, 3)
    r = VOXEL
    dis_voxel = jnp.round(pd / r)
    h = _dense(pd, wp["ie1"])
    h = jax.nn.relu(_bn(h, wp["ie_bn"]))
    h = _dense(h, wp["ie2"])
    pe_raw = ((pd - dis_voxel * r) / r).reshape(-1, 3)
    h = h + _posenc(pe_raw, wp["pec"]).reshape(pd.shape[0], S, -1)
    h = jax.nn.relu(_bn(_dense(h, wp["pm1"]), wp["pm1_bn"], 0.001))
    h = jax.nn.relu(_bn(_dense(h, wp["pm2"]), wp["pm2_bn"], 0.001))
    h = _bn(_dense(h, wp["pm3"]), wp["pm3_bn"], 0.001)
    h = _bn(_dense(h, wp["pm4"]), wp["pm4_bn"], 0.001)
    attn = jax.nn.softmax(h[:, :, 0], axis=-1)
    return attn.reshape(Bq, Nq, S)


def _run_mlp(x, layers):
    for l in layers:
        x = _leaky(_dense(x, l))
    return x


def kernel(xyz1, xyz2, points1, points2, params):
    x1 = xyz1.transpose(0, 2, 1)
    x2 = xyz2.transpose(0, 2, 1)
    f1 = points1.transpose(0, 2, 1)
    f2 = points2.transpose(0, 2, 1)
    idx12, idx21 = _knn_pallas(x1, x2)

    nx2 = _gather(x2, idx12)
    dir12 = nx2 - x1[:, :, None, :]
    gf2 = _gather(f2, idx12)
    gf1 = jnp.broadcast_to(f1[:, :, None, :], gf2.shape)
    c1 = _run_mlp(jnp.concatenate([gf1, gf2], -1), params["mlp1"])
    attn1 = _intra_patch(dir12, params["wn2"])
    cost1 = jnp.sum(attn1[..., None] * c1, axis=2)

    nx1 = _gather(x1, idx21)
    dir21 = nx1 - x2[:, :, None, :]
    gf1b = _gather(f1, idx21)
    gf2b = jnp.broadcast_to(f2[:, :, None, :], gf1b.shape)
    c2 = _run_mlp(jnp.concatenate([gf2b, gf1b], -1), params["mlp2"])
    attn2 = _intra_patch(dir21, params["wn2"])
    cost2 = jnp.sum(attn2[..., None] * c2, axis=2)

    gc2 = _gather(cost2, idx12)
    c3 = _run_mlp(jnp.concatenate([gc2, dir12], -1), params["mlp3"])
    cost21 = jnp.sum(attn1[..., None] * c3, axis=2)

    c4 = _run_mlp(jnp.concatenate([cost1, cost21], -1), params["mlp4"])
    flow = jnp.sum(attn1[..., None] * dir12, axis=2)
    return c4.transpose(0, 2, 1), flow.transpose(0, 2, 1)


# speedup vs baseline: 3.3185x; 3.3185x over previous
import functools
import math

import jax
import jax.numpy as jnp
from jax import lax
from jax.experimental import pallas as pl
from jax.experimental.pallas import tpu as pltpu
from jax.experimental.pallas import tpu_sc as plsc

C_IN = 64
NSAMPLE = 16
VOXEL = 0.25

_QB = 256
_MB = 2048
_PB = _MB // NSAMPLE

_HIGH = lax.Precision.HIGHEST


def _mm(x, wt, precision=None):
    return lax.dot_general(x, wt, (((1,), (0,)), ((), ())),
                           precision=precision,
                           preferred_element_type=jnp.float32)


def _leaky(x):
    return jnp.where(x > 0, x, 0.1 * x)


def _knn_body(qmat_ref, bmat_ref, out_ref):
    q = qmat_ref[0]
    bm = bmat_ref[0]
    s = jnp.dot(q, bm, preferred_element_type=jnp.float32)
    qq = (q[:, 0:1] * q[:, 0:1] + q[:, 1:2] * q[:, 1:2]) + q[:, 2:3] * q[:, 2:3]
    bb = bm[3:4, :]
    d = (qq - 2.0 * s) + bb
    iota = lax.broadcasted_iota(jnp.int32, d.shape, 1)
    big = jnp.int32(2 ** 30)
    for r in range(NSAMPLE):
        m = jnp.min(d, axis=1, keepdims=True)
        sel = jnp.min(jnp.where(d <= m, iota, big), axis=1, keepdims=True)
        out_ref[0, :, r:r + 1] = sel
        d = jnp.where(iota == sel, jnp.float32(jnp.inf), d)


def _knn_pallas(x1, x2):
    B, N, _ = x1.shape
    q_all = jnp.concatenate([x1, x2], axis=0)
    b_all = jnp.concatenate([x2, x1], axis=0)
    zeros1 = jnp.zeros((2 * B, N, 1), jnp.float32)
    zeros4 = jnp.zeros((2 * B, N, 4), jnp.float32)
    qmat = jnp.concatenate([q_all, zeros1, zeros4], axis=-1)
    bb = jnp.sum(b_all * b_all, axis=-1, keepdims=True)
    bmat = jnp.concatenate([b_all, bb, zeros4], axis=-1).transpose(0, 2, 1)

    return pl.pallas_call(
        _knn_body,
        grid=(2 * B, N // _QB),
        in_specs=[
            pl.BlockSpec((1, _QB, 8), lambda g, i: (g, i, 0)),
            pl.BlockSpec((1, 8, N), lambda g, i: (g, 0, 0)),
        ],
        out_specs=pl.BlockSpec((1, _QB, NSAMPLE), lambda g, i: (g, i, 0)),
        out_shape=jax.ShapeDtypeStruct((2 * B, N, NSAMPLE), jnp.int32),
    )(qmat, bmat)


def _gather_rows(table, idx, chunk):
    T, D = table.shape
    Mtot = idx.shape[0]
    NC, NS = 2, 16
    NW = NC * NS
    per_w = Mtot // NW
    n_iter = per_w // chunk
    mesh = plsc.VectorSubcoreMesh(core_axis_name="c", subcore_axis_name="s")

    @functools.partial(
        pl.kernel, mesh=mesh,
        out_type=jax.ShapeDtypeStruct((Mtot, D), jnp.float32),
        scratch_types=[
            pltpu.VMEM((chunk,), jnp.int32),
            pltpu.VMEM((chunk, D), jnp.float32),
            pltpu.SemaphoreType.DMA,
        ],
    )
    def k(table_hbm, idx_hbm, out_hbm, idx_v, rows_v, sem):
        wid = lax.axis_index("s") * NC + lax.axis_index("c")
        base = wid * per_w

        def body(j, carry):
            off = base + j * chunk
            pltpu.sync_copy(idx_hbm.at[pl.ds(off, chunk)], idx_v)
            pltpu.async_copy(table_hbm.at[idx_v], rows_v, sem).wait()
            pltpu.sync_copy(rows_v, out_hbm.at[pl.ds(off, chunk)])
            return carry

        lax.fori_loop(0, n_iter, body, 0)

    return k(table, idx)


def _p1_body(nx_ref, xq_ref, w_ref, b_ref, pd_ref, s_ref, q_ref):
    i = pl.program_id(1)
    xq = xq_ref[...]
    xq_rep = jnp.broadcast_to(xq[:, None, :], (_PB, NSAMPLE, 16)).reshape(_MB, 16)
    pd = nx_ref[:, 0:16] - xq_rep
    pd_ref[...] = pd
    h0 = _mm(pd, w_ref[...]) + b_ref[...]

    @pl.when(i == 0)
    def _():
        s_ref[...] = jnp.zeros_like(s_ref)
        q_ref[...] = jnp.zeros_like(q_ref)

    s_ref[...] += jnp.sum(h0, axis=0, keepdims=True)[None]
    q_ref[...] += jnp.sum(h0 * h0, axis=0, keepdims=True)[None]


def _p2_body(pd_ref, cst_ref, wie1_ref, bie1_ref, wie2_ref, bie2_ref,
             pec_ref, bpec_ref, d24_ref, wpm1_ref, bpm1_ref,
             h3_ref, s_ref, q_ref):
    i = pl.program_id(1)
    pd = pd_ref[...]
    h0 = _mm(pd, wie1_ref[...]) + bie1_ref[...]
    a0 = cst_ref[0, 0:1, 0:64]
    c0 = cst_ref[0, 1:2, 0:64]
    h1 = jax.nn.relu(h0 * a0 + c0)
    h2 = _mm(h1, wie2_ref[...]) + bie2_ref[...]

    r = VOXEL
    dis_voxel = jnp.round(pd / r)
    pe_raw = (pd - dis_voxel * r) / r
    e = pe_raw / (1.0 + 1e-06) * (2.0 * math.pi)
    cols = [jnp.broadcast_to(e[:, c:c + 1], (_MB, 8)) for c in range(3)]
    e24 = jnp.concatenate(cols, axis=1)
    q24 = e24 / d24_ref[...]
    lane = lax.broadcasted_iota(jnp.int32, (_MB, 24), 1)
    feats = jnp.where(lane % 2 == 0, jnp.sin(q24), jnp.cos(q24))
    pos = _mm(feats, pec_ref[...]) + bpec_ref[...]

    h3 = h2 + pos
    h3_ref[...] = h3
    g1 = _mm(h3, wpm1_ref[...]) + bpm1_ref[...]

    @pl.when(i == 0)
    def _():
        s_ref[...] = jnp.zeros_like(s_ref)
        q_ref[...] = jnp.zeros_like(q_ref)

    s_ref[...] += jnp.sum(g1, axis=0, keepdims=True)[None]
    q_ref[...] += jnp.sum(g1 * g1, axis=0, keepdims=True)[None]


def _h4_from_h3(h3, cst_ref, wpm1_ref, bpm1_ref):
    g1 = _mm(h3, wpm1_ref[...]) + bpm1_ref[...]
    a1 = cst_ref[0, 0:1, 0:64]
    c1 = cst_ref[0, 1:2, 0:64]
    return jax.nn.relu(g1 * a1 + c1)


def _p3_body(h3_ref, cst_ref, wpm1_ref, bpm1_ref, s_ref, m_ref):
    i = pl.program_id(1)
    h4 = _h4_from_h3(h3_ref[...], cst_ref, wpm1_ref, bpm1_ref)

    @pl.when(i == 0)
    def _():
        s_ref[...] = jnp.zeros_like(s_ref)
        m_ref[...] = jnp.zeros_like(m_ref)

    s_ref[...] += jnp.sum(h4, axis=0, keepdims=True)[None]
    mom = lax.dot_general(h4, h4, (((0,), (0,)), ((), ())),
                          precision=_HIGH, preferred_element_type=jnp.float32)
    m_ref[...] += mom[None]


def _h5_from_h3(h3, cst_ref, wpm1_ref, bpm1_ref, wpm2_ref, bpm2_ref):
    h4 = _h4_from_h3(h3, cst_ref, wpm1_ref, bpm1_ref)
    g2 = _mm(h4, wpm2_ref[...]) + bpm2_ref[...]
    a2 = cst_ref[0, 2:3, 0:64]
    c2 = cst_ref[0, 3:4, 0:64]
    return jax.nn.relu(g2 * a2 + c2)


def _p4_body(h3_ref, cst_ref, wpm1_ref, bpm1_ref, wpm2_ref, bpm2_ref,
             s_ref, m_ref):
    i = pl.program_id(1)
    h5 = _h5_from_h3(h3_ref[...], cst_ref, wpm1_ref, bpm1_ref, wpm2_ref, bpm2_ref)

    @pl.when(i == 0)
    def _():
        s_ref[...] = jnp.zeros_like(s_ref)
        m_ref[...] = jnp.zeros_like(m_ref)

    s_ref[...] += jnp.sum(h5, axis=0, keepdims=True)[None]
    mom = lax.dot_general(h5, h5, (((0,), (0,)), ((), ())),
                          precision=_HIGH, preferred_element_type=jnp.float32)
    m_ref[...] += mom[None]


def _attn_from_h3(h3, cst_ref, wpm1_ref, bpm1_ref, wpm2_ref, bpm2_ref,
                  wpm3_ref, bpm3_ref):
    h5 = _h5_from_h3(h3, cst_ref, wpm1_ref, bpm1_ref, wpm2_ref, bpm2_ref)
    g3 = _mm(h5, wpm3_ref[...]) + bpm3_ref[...]
    a3 = cst_ref[0, 4:5, 0:32]
    c3 = cst_ref[0, 5:6, 0:32]
    h6 = g3 * a3 + c3
    w4 = cst_ref[0, 6:7, 0:32]
    b4 = cst_ref[0, 7:8, 0:1]
    a4 = cst_ref[0, 7:8, 1:2]
    c4 = cst_ref[0, 7:8, 2:3]
    g4 = jnp.sum(h6 * w4, axis=1, keepdims=True) + b4
    h7 = g4 * a4 + c4
    att = h7.reshape(_PB, NSAMPLE)
    mx = jnp.max(att, axis=1, keepdims=True)
    ex = jnp.exp(att - mx)
    return ex / jnp.sum(ex, axis=1, keepdims=True)


def _feat_mlp(fq, gfb, wq_ref, wg_ref, b1_ref, w2_ref, b2_ref, w3_ref, b3_ref):
    uq = _mm(fq, wq_ref[...])
    urep = jnp.broadcast_to(uq[:, None, :], (_PB, NSAMPLE, 64)).reshape(_MB, 64)
    t1 = _leaky(urep + _mm(gfb[:, 64:128], wg_ref[...]) + b1_ref[...])
    t2 = _leaky(_mm(t1, w2_ref[...]) + b2_ref[...])
    return _leaky(_mm(t2, w3_ref[...]) + b3_ref[...])


def _wsum(att, x, d):
    return jnp.sum(att[:, :, None] * x.reshape(_PB, NSAMPLE, d), axis=1)


def _p5a_body(h3_ref, cst_ref, wpm1_ref, bpm1_ref, wpm2_ref, bpm2_ref,
              wpm3_ref, bpm3_ref, fq_ref, gfb_ref,
              wq_ref, wg_ref, b1_ref, w2_ref, b2_ref, w3_ref, b3_ref,
              cost_ref):
    att = _attn_from_h3(h3_ref[...], cst_ref, wpm1_ref, bpm1_ref,
                        wpm2_ref, bpm2_ref, wpm3_ref, bpm3_ref)
    c = _feat_mlp(fq_ref[...], gfb_ref[...], wq_ref, wg_ref, b1_ref,
                  w2_ref, b2_ref, w3_ref, b3_ref)
    cost_ref[...] = _wsum(att, c, 128)


def _p5b_body(h3_ref, cst_ref, wpm1_ref, bpm1_ref, wpm2_ref, bpm2_ref,
              wpm3_ref, bpm3_ref, fq_ref, gfb_ref,
              wq_ref, wg_ref, b1_ref, w2_ref, b2_ref, w3_ref, b3_ref,
              pd_ref, gc2_ref, w3a_ref, w3b_ref, b31_ref, w32_ref, b32_ref,
              w33_ref, b33_ref,
              w4a_ref, w4b_ref, b41_ref, w42_ref, b42_ref, w43_ref, b43_ref,
              c4_ref, flow_ref):
    att = _attn_from_h3(h3_ref[...], cst_ref, wpm1_ref, bpm1_ref,
                        wpm2_ref, bpm2_ref, wpm3_ref, bpm3_ref)
    c1 = _feat_mlp(fq_ref[...], gfb_ref[...], wq_ref, wg_ref, b1_ref,
                   w2_ref, b2_ref, w3_ref, b3_ref)
    cost1 = _wsum(att, c1, 128)

    pd = pd_ref[...]
    t = _leaky(_mm(gc2_ref[...], w3a_ref[...]) + _mm(pd, w3b_ref[...]) + b31_ref[...])
    t2 = _leaky(_mm(t, w32_ref[...]) + b32_ref[...])
    c3 = _leaky(_mm(t2, w33_ref[...]) + b33_ref[...])
    cost21 = _wsum(att, c3, 128)

    flow_ref[...] = _wsum(att, pd, 16)

    m1 = _leaky(_mm(cost1, w4a_ref[...]) + _mm(cost21, w4b_ref[...]) + b41_ref[...])
    m2 = _leaky(_mm(m1, w42_ref[...]) + b42_ref[...])
    c4_ref[...] = _leaky(_mm(m2, w43_ref[...]) + b43_ref[...])


def _row(v, width=128):
    v = jnp.asarray(v, jnp.float32).reshape(1, -1)
    return jnp.pad(v, ((0, 0), (0, width - v.shape[1])))


def _bn_ac(mean, var, g, b, eps):
    a = g / jnp.sqrt(var + eps)
    return a, b - mean * a


def _stats_from_sums(s, q, count):
    mean = s / count
    var = q / count - mean * mean
    return mean, var


def _lin_stats(mean_x, mom_x, w, b):
    mean_y = mean_x @ w.T + b
    wm = w @ mom_x
    e2 = jnp.sum(wm * w, axis=1) + 2.0 * b * (w @ mean_x) + b * b
    return mean_y, e2 - mean_y * mean_y


def kernel(xyz1, xyz2, points1, points2, params):
    B = xyz1.shape[0]
    N = xyz1.shape[2]
    K = NSAMPLE
    M_per_g = N * K
    n_mb = M_per_g // _MB
    Mdir = jnp.float32(B * N * K)

    x1 = xyz1.transpose(0, 2, 1)
    x2 = xyz2.transpose(0, 2, 1)
    f1 = points1.transpose(0, 2, 1)
    f2 = points2.transpose(0, 2, 1)

    idx_all = _knn_pallas(x1, x2)

    pad13 = jnp.zeros((2 * B * N, 13), jnp.float32)
    base_coords = jnp.concatenate(
        [x2.reshape(B * N, 3), x1.reshape(B * N, 3)], axis=0)
    base_feats = jnp.concatenate(
        [f2.reshape(B * N, C_IN), f1.reshape(B * N, C_IN)], axis=0)
    t_base = jnp.concatenate(
        [base_coords, jnp.zeros((2 * B * N, 61), jnp.float32), base_feats],
        axis=1)

    offs = (jnp.arange(2 * B, dtype=jnp.int32) * N)[:, None, None]
    idx_flat = (idx_all + offs).reshape(-1)

    gall = _gather_rows(t_base, idx_flat, 512)

    q_pad = jnp.concatenate(
        [jnp.concatenate([x1.reshape(B * N, 3), x2.reshape(B * N, 3)], axis=0),
         pad13], axis=1)
    fq = jnp.concatenate([f1.reshape(B * N, C_IN), f2.reshape(B * N, C_IN)],
                         axis=0)

    wp = params["wn2"]
    Mall = 2 * B * N * K

    wie1_p = jnp.pad(wp["ie1"]["W"].T, ((0, 13), (0, 128 - 64)))
    p1 = pl.pallas_call(
        _p1_body,
        grid=(2 * B, n_mb),
        in_specs=[
            pl.BlockSpec((_MB, 128), lambda g, i: (g * n_mb + i, 0)),
            pl.BlockSpec((_PB, 16), lambda g, i: (g * n_mb + i, 0)),
            pl.BlockSpec((16, 128), lambda g, i: (0, 0)),
            pl.BlockSpec((1, 128), lambda g, i: (0, 0)),
        ],
        out_specs=[
            pl.BlockSpec((_MB, 16), lambda g, i: (g * n_mb + i, 0)),
            pl.BlockSpec((1, 1, 128), lambda g, i: (g, 0, 0)),
            pl.BlockSpec((1, 1, 128), lambda g, i: (g, 0, 0)),
        ],
        out_shape=[
            jax.ShapeDtypeStruct((Mall, 16), jnp.float32),
            jax.ShapeDtypeStruct((2 * B, 1, 128), jnp.float32),
            jax.ShapeDtypeStruct((2 * B, 1, 128), jnp.float32),
        ],
    )(gall, q_pad, wie1_p, _row(wp["ie1"]["b"]))
    pd_all, s0, q0 = p1
    s0 = s0[:, 0, :64].reshape(2, B, 64).sum(1)
    q0 = q0[:, 0, :64].reshape(2, B, 64).sum(1)
    mean0, var0 = _stats_from_sums(s0, q0, Mdir)
    a0, c0 = _bn_ac(mean0, var0, wp["ie_bn"]["g"], wp["ie_bn"]["b"], 1e-05)

    def _cst2(a_list):
        rows = []
        for d in range(2):
            rr = [_row(r[d]) if r.ndim == 2 else _row(r) for r in a_list]
            while len(rr) < 8:
                rr.append(jnp.zeros((1, 128), jnp.float32))
            rows.append(jnp.concatenate(rr, axis=0)[None])
        return jnp.concatenate(rows, axis=0)

    cst_p2 = _cst2([a0, c0])

    d24 = jnp.array([[1.0, 1.0, 10.0, 10.0, 100.0, 100.0, 1000.0, 1000.0] * 3],
                    jnp.float32)
    wie1_t = jnp.pad(wp["ie1"]["W"].T, ((0, 13), (0, 0)))
    wpm1_p = jnp.pad(wp["pm1"]["W"].T, ((0, 0), (0, 64)))
    p2 = pl.pallas_call(
        _p2_body,
        grid=(2 * B, n_mb),
        in_specs=[
            pl.BlockSpec((_MB, 16), lambda g, i: (g * n_mb + i, 0)),
            pl.BlockSpec((1, 8, 128), lambda g, i: (g // 2, 0, 0)),
            pl.BlockSpec((16, 64), lambda g, i: (0, 0)),
            pl.BlockSpec((1, 64), lambda g, i: (0, 0)),
            pl.BlockSpec((64, 64), lambda g, i: (0, 0)),
            pl.BlockSpec((1, 64), lambda g, i: (0, 0)),
            pl.BlockSpec((24, 64), lambda g, i: (0, 0)),
            pl.BlockSpec((1, 64), lambda g, i: (0, 0)),
            pl.BlockSpec((1, 24), lambda g, i: (0, 0)),
            pl.BlockSpec((64, 128), lambda g, i: (0, 0)),
            pl.BlockSpec((1, 128), lambda g, i: (0, 0)),
        ],
        out_specs=[
            pl.BlockSpec((_MB, 64), lambda g, i: (g * n_mb + i, 0)),
            pl.BlockSpec((1, 1, 128), lambda g, i: (g, 0, 0)),
            pl.BlockSpec((1, 1, 128), lambda g, i: (g, 0, 0)),
        ],
        out_shape=[
            jax.ShapeDtypeStruct((Mall, 64), jnp.float32),
            jax.ShapeDtypeStruct((2 * B, 1, 128), jnp.float32),
            jax.ShapeDtypeStruct((2 * B, 1, 128), jnp.float32),
        ],
    )(pd_all, cst_p2, wie1_t, _row(wp["ie1"]["b"], 64),
      wp["ie2"]["W"].T, _row(wp["ie2"]["b"], 64),
      wp["pec"]["W"].T, _row(wp["pec"]["b"], 64), d24,
      wpm1_p, _row(wp["pm1"]["b"]))
    h3_all, s1, q1 = p2
    s1 = s1[:, 0, :64].reshape(2, B, 64).sum(1)
    q1 = q1[:, 0, :64].reshape(2, B, 64).sum(1)
    mean1, var1 = _stats_from_sums(s1, q1, Mdir)
    a1, c1 = _bn_ac(mean1, var1, wp["pm1_bn"]["g"], wp["pm1_bn"]["b"], 0.001)

    cst_p3 = _cst2([a1, c1])
    wpm1_t = wp["pm1"]["W"].T
    bpm1 = _row(wp["pm1"]["b"], 64)

    p3 = pl.pallas_call(
        _p3_body,
        grid=(2 * B, n_mb),
        in_specs=[
            pl.BlockSpec((_MB, 64), lambda g, i: (g * n_mb + i, 0)),
            pl.BlockSpec((1, 8, 128), lambda g, i: (g // 2, 0, 0)),
            pl.BlockSpec((64, 64), lambda g, i: (0, 0)),
            pl.BlockSpec((1, 64), lambda g, i: (0, 0)),
        ],
        out_specs=[
            pl.BlockSpec((1, 1, 64), lambda g, i: (g, 0, 0)),
            pl.BlockSpec((1, 64, 64), lambda g, i: (g, 0, 0)),
        ],
        out_shape=[
            jax.ShapeDtypeStruct((2 * B, 1, 64), jnp.float32),
            jax.ShapeDtypeStruct((2 * B, 64, 64), jnp.float32),
        ],
    )(h3_all, cst_p3, wpm1_t, bpm1)
    s4, m4 = p3
    mean_h4 = s4[:, 0, :].reshape(2, B, 64).sum(1) / Mdir
    mom_h4 = m4.reshape(2, B, 64, 64).sum(1) / Mdir
    mean2 = jnp.stack([_lin_stats(mean_h4[d], mom_h4[d], wp["pm2"]["W"],
                                  wp["pm2"]["b"])[0] for d in range(2)])
    var2 = jnp.stack([_lin_stats(mean_h4[d], mom_h4[d], wp["pm2"]["W"],
                                 wp["pm2"]["b"])[1] for d in range(2)])
    a2, c2 = _bn_ac(mean2, var2, wp["pm2_bn"]["g"], wp["pm2_bn"]["b"], 0.001)

    cst_p4 = _cst2([a1, c1, a2, c2])
    wpm2_t = wp["pm2"]["W"].T
    bpm2 = _row(wp["pm2"]["b"], 64)

    p4 = pl.pallas_call(
        _p4_body,
        grid=(2 * B, n_mb),
        in_specs=[
            pl.BlockSpec((_MB, 64), lambda g, i: (g * n_mb + i, 0)),
            pl.BlockSpec((1, 8, 128), lambda g, i: (g // 2, 0, 0)),
            pl.BlockSpec((64, 64), lambda g, i: (0, 0)),
            pl.BlockSpec((1, 64), lambda g, i: (0, 0)),
            pl.BlockSpec((64, 64), lambda g, i: (0, 0)),
            pl.BlockSpec((1, 64), lambda g, i: (0, 0)),
        ],
        out_specs=[
            pl.BlockSpec((1, 1, 64), lambda g, i: (g, 0, 0)),
            pl.BlockSpec((1, 64, 64), lambda g, i: (g, 0, 0)),
        ],
        out_shape=[
            jax.ShapeDtypeStruct((2 * B, 1, 64), jnp.float32),
            jax.ShapeDtypeStruct((2 * B, 64, 64), jnp.float32),
        ],
    )(h3_all, cst_p4, wpm1_t, bpm1, wpm2_t, bpm2)
    s5, m5 = p4
    mean_h5 = s5[:, 0, :].reshape(2, B, 64).sum(1) / Mdir
    mom_h5 = m5.reshape(2, B, 64, 64).sum(1) / Mdir

    a3l, c3l, w4r, sc_rows = [], [], [], []
    for d in range(2):
        m3, v3 = _lin_stats(mean_h5[d], mom_h5[d], wp["pm3"]["W"], wp["pm3"]["b"])
        a3, c3 = _bn_ac(m3, v3, wp["pm3_bn"]["g"], wp["pm3_bn"]["b"], 0.001)
        w4row = wp["pm4"]["W"][0]
        w_eff = wp["pm3"]["W"].T @ (a3 * w4row)
        b_eff = (wp["pm3"]["b"] * (a3 * w4row)).sum() + (c3 * w4row).sum() \
            + wp["pm4"]["b"][0]
        mean4l = mean_h5[d] @ w_eff + b_eff
        e4 = w_eff @ mom_h5[d] @ w_eff + 2.0 * b_eff * (mean_h5[d] @ w_eff) \
            + b_eff * b_eff
        var4l = e4 - mean4l * mean4l
        a4 = wp["pm4_bn"]["g"][0] / jnp.sqrt(var4l + 0.001)
        c4bn = wp["pm4_bn"]["b"][0] - mean4l * a4
        a3l.append(a3)
        c3l.append(c3)
        w4r.append(w4row)
        sc_rows.append(jnp.stack([wp["pm4"]["b"][0], a4, c4bn]))

    def _cst_dir(d):
        rows = [_row(a1[d]), _row(c1[d]), _row(a2[d]), _row(c2[d]),
                _row(a3l[d]), _row(c3l[d]), _row(w4r[d]), _row(sc_rows[d])]
        return jnp.concatenate(rows, axis=0)[None]

    cst5 = jnp.concatenate([_cst_dir(0), _cst_dir(1)], axis=0)

    wpm3_t = wp["pm3"]["W"].T
    bpm3 = _row(wp["pm3"]["b"], 32)

    def _mlp_w(ml):
        w1, w2, w3 = ml
        return (w1["W"][:, :64].T, w1["W"][:, 64:].T, _row(w1["b"], 64),
                w2["W"].T, _row(w2["b"], 64), w3["W"].T, _row(w3["b"], 128))

    h3_cc = h3_all.reshape(2, B * N * K, 64)
    gfb_cc = gall.reshape(2, B * N * K, 128)
    fq_cc = fq.reshape(2, B * N, 64)
    nb2 = B * M_per_g // _MB

    attn_specs = [
        pl.BlockSpec((_MB, 64), lambda i: (i, 0)),
        pl.BlockSpec((1, 8, 128), lambda i: (0, 0, 0)),
        pl.BlockSpec((64, 64), lambda i: (0, 0)),
        pl.BlockSpec((1, 64), lambda i: (0, 0)),
        pl.BlockSpec((64, 64), lambda i: (0, 0)),
        pl.BlockSpec((1, 64), lambda i: (0, 0)),
        pl.BlockSpec((64, 32), lambda i: (0, 0)),
        pl.BlockSpec((1, 32), lambda i: (0, 0)),
        pl.BlockSpec((_PB, 64), lambda i: (i, 0)),
        pl.BlockSpec((_MB, 128), lambda i: (i, 0)),
        pl.BlockSpec((64, 64), lambda i: (0, 0)),
        pl.BlockSpec((64, 64), lambda i: (0, 0)),
        pl.BlockSpec((1, 64), lambda i: (0, 0)),
        pl.BlockSpec((64, 64), lambda i: (0, 0)),
        pl.BlockSpec((1, 64), lambda i: (0, 0)),
        pl.BlockSpec((64, 128), lambda i: (0, 0)),
        pl.BlockSpec((1, 128), lambda i: (0, 0)),
    ]

    cost2 = pl.pallas_call(
        _p5a_body,
        grid=(nb2,),
        in_specs=list(attn_specs),
        out_specs=pl.BlockSpec((_PB, 128), lambda i: (i, 0)),
        out_shape=jax.ShapeDtypeStruct((B * N, 128), jnp.float32),
    )(h3_cc[1], cst5[1:2], wpm1_t, bpm1, wpm2_t, bpm2, wpm3_t, bpm3,
      fq_cc[1], gfb_cc[1], *_mlp_w(params["mlp2"]))

    idx12_flat = idx_flat[:B * N * K]
    gc2 = _gather_rows(cost2, idx12_flat, 512)

    w3 = params["mlp3"]
    w4 = params["mlp4"]
    w3b_p = jnp.pad(w3[0]["W"][:, 128:].T, ((0, 13), (0, 0)))
    c4_rows, flow_pad = pl.pallas_call(
        _p5b_body,
        grid=(nb2,),
        in_specs=list(attn_specs) + [
            pl.BlockSpec((_MB, 16), lambda i: (i, 0)),
            pl.BlockSpec((_MB, 128), lambda i: (i, 0)),
            pl.BlockSpec((128, 64), lambda i: (0, 0)),
            pl.BlockSpec((16, 64), lambda i: (0, 0)),
            pl.BlockSpec((1, 64), lambda i: (0, 0)),
            pl.BlockSpec((64, 64), lambda i: (0, 0)),
            pl.BlockSpec((1, 64), lambda i: (0, 0)),
            pl.BlockSpec((64, 128), lambda i: (0, 0)),
            pl.BlockSpec((1, 128), lambda i: (0, 0)),
            pl.BlockSpec((128, 64), lambda i: (0, 0)),
            pl.BlockSpec((128, 64), lambda i: (0, 0)),
            pl.BlockSpec((1, 64), lambda i: (0, 0)),
            pl.BlockSpec((64, 64), lambda i: (0, 0)),
            pl.BlockSpec((1, 64), lambda i: (0, 0)),
            pl.BlockSpec((64, 128), lambda i: (0, 0)),
            pl.BlockSpec((1, 128), lambda i: (0, 0)),
        ],
        out_specs=[
            pl.BlockSpec((_PB, 128), lambda i: (i, 0)),
            pl.BlockSpec((_PB, 16), lambda i: (i, 0)),
        ],
        out_shape=[
            jax.ShapeDtypeStruct((B * N, 128), jnp.float32),
            jax.ShapeDtypeStruct((B * N, 16), jnp.float32),
        ],
    )(h3_cc[0], cst5[0:1], wpm1_t, bpm1, wpm2_t, bpm2, wpm3_t, bpm3,
      fq_cc[0], gfb_cc[0], *_mlp_w(params["mlp1"]),
      pd_all[:B * N * K], gc2,
      w3[0]["W"][:, :128].T, w3b_p, _row(w3[0]["b"], 64),
      w3[1]["W"].T, _row(w3[1]["b"], 64), w3[2]["W"].T, _row(w3[2]["b"], 128),
      w4[0]["W"][:, :128].T, w4[0]["W"][:, 128:].T, _row(w4[0]["b"], 64),
      w4[1]["W"].T, _row(w4[1]["b"], 64), w4[2]["W"].T, _row(w4[2]["b"], 128))

    c4 = c4_rows.reshape(B, N, 128).transpose(0, 2, 1)
    flow = flow_pad[:, :3].reshape(B, N, 3).transpose(0, 2, 1)
    return c4, flow


# speedup vs baseline: 3.5359x; 1.0655x over previous
import functools
import math

import jax
import jax.numpy as jnp
from jax import lax
from jax.experimental import pallas as pl
from jax.experimental.pallas import tpu as pltpu
from jax.experimental.pallas import tpu_sc as plsc

C_IN = 64
NSAMPLE = 16
VOXEL = 0.25

_QB = 256
_MB = 2048
_PB = _MB // NSAMPLE

_HIGH = lax.Precision.HIGHEST


def _mm(x, wt, precision=None):
    return lax.dot_general(x, wt, (((1,), (0,)), ((), ())),
                           precision=precision,
                           preferred_element_type=jnp.float32)


def _leaky(x):
    return jnp.where(x > 0, x, 0.1 * x)


def _knn_body(qmat_ref, bmat_ref, out_ref):
    q = qmat_ref[0]
    bm = bmat_ref[0]
    s = jnp.dot(q, bm, preferred_element_type=jnp.float32)
    qq = (q[:, 0:1] * q[:, 0:1] + q[:, 1:2] * q[:, 1:2]) + q[:, 2:3] * q[:, 2:3]
    bb = bm[3:4, :]
    d = (qq - 2.0 * s) + bb
    iota = lax.broadcasted_iota(jnp.int32, d.shape, 1)
    big = jnp.int32(2 ** 30)
    for r in range(NSAMPLE):
        m = jnp.min(d, axis=1, keepdims=True)
        sel = jnp.min(jnp.where(d <= m, iota, big), axis=1, keepdims=True)
        out_ref[0, :, r:r + 1] = sel
        d = jnp.where(iota == sel, jnp.float32(jnp.inf), d)


def _knn_pallas(x1, x2):
    B, N, _ = x1.shape
    q_all = jnp.concatenate([x1, x2], axis=0)
    b_all = jnp.concatenate([x2, x1], axis=0)
    zeros1 = jnp.zeros((2 * B, N, 1), jnp.float32)
    zeros4 = jnp.zeros((2 * B, N, 4), jnp.float32)
    qmat = jnp.concatenate([q_all, zeros1, zeros4], axis=-1)
    bb = jnp.sum(b_all * b_all, axis=-1, keepdims=True)
    bmat = jnp.concatenate([b_all, bb, zeros4], axis=-1).transpose(0, 2, 1)

    return pl.pallas_call(
        _knn_body,
        grid=(2 * B, N // _QB),
        in_specs=[
            pl.BlockSpec((1, _QB, 8), lambda g, i: (g, i, 0)),
            pl.BlockSpec((1, 8, N), lambda g, i: (g, 0, 0)),
        ],
        out_specs=pl.BlockSpec((1, _QB, NSAMPLE), lambda g, i: (g, i, 0)),
        out_shape=jax.ShapeDtypeStruct((2 * B, N, NSAMPLE), jnp.int32),
    )(qmat, bmat)


def _gather_rows(table, idx, chunk):
    T, D = table.shape
    Mtot = idx.shape[0]
    NC, NS = 2, 16
    NW = NC * NS
    per_w = Mtot // NW
    n_iter = per_w // chunk
    mesh = plsc.VectorSubcoreMesh(core_axis_name="c", subcore_axis_name="s")

    @functools.partial(
        pl.kernel, mesh=mesh,
        out_type=jax.ShapeDtypeStruct((Mtot, D), jnp.float32),
        scratch_types=[
            pltpu.VMEM((chunk,), jnp.int32),
            pltpu.VMEM((chunk, D), jnp.float32),
            pltpu.SemaphoreType.DMA,
        ],
    )
    def k(table_hbm, idx_hbm, out_hbm, idx_v, rows_v, sem):
        wid = lax.axis_index("s") * NC + lax.axis_index("c")
        base = wid * per_w

        def body(j, carry):
            off = base + j * chunk
            pltpu.sync_copy(idx_hbm.at[pl.ds(off, chunk)], idx_v)
            pltpu.async_copy(table_hbm.at[idx_v], rows_v, sem).wait()
            pltpu.sync_copy(rows_v, out_hbm.at[pl.ds(off, chunk)])
            return carry

        lax.fori_loop(0, n_iter, body, 0)

    return k(table, idx)


def _p1_body(nx_ref, xq_ref, w_ref, b_ref, pd_ref, s_ref, q_ref):
    i = pl.program_id(1)
    xq = xq_ref[...]
    xq_rep = jnp.broadcast_to(xq[:, None, :], (_PB, NSAMPLE, 16)).reshape(_MB, 16)
    pd = nx_ref[:, 0:16] - xq_rep
    pd_ref[...] = pd
    h0 = _mm(pd, w_ref[...]) + b_ref[...]

    @pl.when(i == 0)
    def _():
        s_ref[...] = jnp.zeros_like(s_ref)
        q_ref[...] = jnp.zeros_like(q_ref)

    s_ref[...] += jnp.sum(h0, axis=0, keepdims=True)[None]
    q_ref[...] += jnp.sum(h0 * h0, axis=0, keepdims=True)[None]


def _p2_body(pd_ref, cst_ref, wie1_ref, bie1_ref, wie2_ref, bie2_ref,
             pec_ref, bpec_ref, d24_ref, wpm1_ref, bpm1_ref,
             h3_ref, s_ref, q_ref):
    i = pl.program_id(1)
    pd = pd_ref[...]
    h0 = _mm(pd, wie1_ref[...]) + bie1_ref[...]
    a0 = cst_ref[0, 0:1, 0:64]
    c0 = cst_ref[0, 1:2, 0:64]
    h1 = jax.nn.relu(h0 * a0 + c0)
    h2 = _mm(h1, wie2_ref[...]) + bie2_ref[...]

    r = VOXEL
    dis_voxel = jnp.round(pd / r)
    pe_raw = (pd - dis_voxel * r) / r
    e = pe_raw / (1.0 + 1e-06) * (2.0 * math.pi)
    cols = [jnp.broadcast_to(e[:, c:c + 1], (_MB, 8)) for c in range(3)]
    e24 = jnp.concatenate(cols, axis=1)
    q24 = e24 / d24_ref[...]
    lane = lax.broadcasted_iota(jnp.int32, (_MB, 24), 1)
    feats = jnp.where(lane % 2 == 0, jnp.sin(q24), jnp.cos(q24))
    pos = _mm(feats, pec_ref[...]) + bpec_ref[...]

    h3 = h2 + pos
    h3_ref[...] = h3
    g1 = _mm(h3, wpm1_ref[...]) + bpm1_ref[...]

    @pl.when(i == 0)
    def _():
        s_ref[...] = jnp.zeros_like(s_ref)
        q_ref[...] = jnp.zeros_like(q_ref)

    s_ref[...] += jnp.sum(g1, axis=0, keepdims=True)[None]
    q_ref[...] += jnp.sum(g1 * g1, axis=0, keepdims=True)[None]


def _h4_from_h3(h3, cst_ref, wpm1_ref, bpm1_ref):
    g1 = _mm(h3, wpm1_ref[...]) + bpm1_ref[...]
    a1 = cst_ref[0, 0:1, 0:64]
    c1 = cst_ref[0, 1:2, 0:64]
    return jax.nn.relu(g1 * a1 + c1)


def _p3_body(h3_ref, cst_ref, wpm1_ref, bpm1_ref, s_ref, m_ref):
    i = pl.program_id(1)
    h4 = _h4_from_h3(h3_ref[...], cst_ref, wpm1_ref, bpm1_ref)

    @pl.when(i == 0)
    def _():
        s_ref[...] = jnp.zeros_like(s_ref)
        m_ref[...] = jnp.zeros_like(m_ref)

    s_ref[...] += jnp.sum(h4, axis=0, keepdims=True)[None]
    mom = lax.dot_general(h4, h4, (((0,), (0,)), ((), ())),
                          precision=_HIGH, preferred_element_type=jnp.float32)
    m_ref[...] += mom[None]


def _h5_from_h3(h3, cst_ref, wpm1_ref, bpm1_ref, wpm2_ref, bpm2_ref):
    h4 = _h4_from_h3(h3, cst_ref, wpm1_ref, bpm1_ref)
    g2 = _mm(h4, wpm2_ref[...]) + bpm2_ref[...]
    a2 = cst_ref[0, 2:3, 0:64]
    c2 = cst_ref[0, 3:4, 0:64]
    return jax.nn.relu(g2 * a2 + c2)


def _p4_body(h3_ref, cst_ref, wpm1_ref, bpm1_ref, wpm2_ref, bpm2_ref,
             s_ref, m_ref):
    i = pl.program_id(1)
    h5 = _h5_from_h3(h3_ref[...], cst_ref, wpm1_ref, bpm1_ref, wpm2_ref, bpm2_ref)

    @pl.when(i == 0)
    def _():
        s_ref[...] = jnp.zeros_like(s_ref)
        m_ref[...] = jnp.zeros_like(m_ref)

    s_ref[...] += jnp.sum(h5, axis=0, keepdims=True)[None]
    mom = lax.dot_general(h5, h5, (((0,), (0,)), ((), ())),
                          precision=_HIGH, preferred_element_type=jnp.float32)
    m_ref[...] += mom[None]


def _attn_from_h3(h3, cst_ref, wpm1_ref, bpm1_ref, wpm2_ref, bpm2_ref,
                  wpm3_ref, bpm3_ref):
    h5 = _h5_from_h3(h3, cst_ref, wpm1_ref, bpm1_ref, wpm2_ref, bpm2_ref)
    g3 = _mm(h5, wpm3_ref[...]) + bpm3_ref[...]
    a3 = cst_ref[0, 4:5, 0:32]
    c3 = cst_ref[0, 5:6, 0:32]
    h6 = g3 * a3 + c3
    w4 = cst_ref[0, 6:7, 0:32]
    b4 = cst_ref[0, 7:8, 0:1]
    a4 = cst_ref[0, 7:8, 1:2]
    c4 = cst_ref[0, 7:8, 2:3]
    g4 = jnp.sum(h6 * w4, axis=1, keepdims=True) + b4
    h7 = g4 * a4 + c4
    att = h7.reshape(_PB, NSAMPLE)
    mx = jnp.max(att, axis=1, keepdims=True)
    ex = jnp.exp(att - mx)
    return ex / jnp.sum(ex, axis=1, keepdims=True)


def _feat_mlp(fq, gfb, wq_ref, wg_ref, b1_ref, w2_ref, b2_ref, w3_ref, b3_ref):
    uq = _mm(fq, wq_ref[...])
    urep = jnp.broadcast_to(uq[:, None, :], (_PB, NSAMPLE, 64)).reshape(_MB, 64)
    t1 = _leaky(urep + _mm(gfb[:, 64:128], wg_ref[...]) + b1_ref[...])
    t2 = _leaky(_mm(t1, w2_ref[...]) + b2_ref[...])
    return _leaky(_mm(t2, w3_ref[...]) + b3_ref[...])


def _wsum(att, x, d):
    return jnp.sum(att[:, :, None] * x.reshape(_PB, NSAMPLE, d), axis=1)


def _p5a_body(h3_ref, cst_ref, wpm1_ref, bpm1_ref, wpm2_ref, bpm2_ref,
              wpm3_ref, bpm3_ref, fq_ref, gfb_ref,
              wq_ref, wg_ref, b1_ref, w2_ref, b2_ref, w3_ref, b3_ref,
              cost_ref):
    att = _attn_from_h3(h3_ref[...], cst_ref, wpm1_ref, bpm1_ref,
                        wpm2_ref, bpm2_ref, wpm3_ref, bpm3_ref)
    c = _feat_mlp(fq_ref[...], gfb_ref[...], wq_ref, wg_ref, b1_ref,
                  w2_ref, b2_ref, w3_ref, b3_ref)
    cost_ref[...] = _wsum(att, c, 128)


def _p5b_body(h3_ref, cst_ref, wpm1_ref, bpm1_ref, wpm2_ref, bpm2_ref,
              wpm3_ref, bpm3_ref, fq_ref, gfb_ref,
              wq_ref, wg_ref, b1_ref, w2_ref, b2_ref, w3_ref, b3_ref,
              pd_ref, gc2_ref, w3a_ref, w3b_ref, b31_ref, w32_ref, b32_ref,
              w33_ref, b33_ref,
              w4a_ref, w4b_ref, b41_ref, w42_ref, b42_ref, w43_ref, b43_ref,
              c4_ref, flow_ref):
    att = _attn_from_h3(h3_ref[...], cst_ref, wpm1_ref, bpm1_ref,
                        wpm2_ref, bpm2_ref, wpm3_ref, bpm3_ref)
    c1 = _feat_mlp(fq_ref[...], gfb_ref[...], wq_ref, wg_ref, b1_ref,
                   w2_ref, b2_ref, w3_ref, b3_ref)
    cost1 = _wsum(att, c1, 128)

    pd = pd_ref[...]
    t = _leaky(_mm(gc2_ref[...], w3a_ref[...]) + _mm(pd, w3b_ref[...]) + b31_ref[...])
    t2 = _leaky(_mm(t, w32_ref[...]) + b32_ref[...])
    c3 = _leaky(_mm(t2, w33_ref[...]) + b33_ref[...])
    cost21 = _wsum(att, c3, 128)

    flow_ref[...] = _wsum(att, pd, 16)

    m1 = _leaky(_mm(cost1, w4a_ref[...]) + _mm(cost21, w4b_ref[...]) + b41_ref[...])
    m2 = _leaky(_mm(m1, w42_ref[...]) + b42_ref[...])
    c4_ref[...] = _leaky(_mm(m2, w43_ref[...]) + b43_ref[...])


def _row(v, width=128):
    v = jnp.asarray(v, jnp.float32).reshape(1, -1)
    return jnp.pad(v, ((0, 0), (0, width - v.shape[1])))


def _bn_ac(mean, var, g, b, eps):
    a = g / jnp.sqrt(var + eps)
    return a, b - mean * a


def _stats_from_sums(s, q, count):
    mean = s / count
    var = q / count - mean * mean
    return mean, var


def _lin_stats(mean_x, mom_x, w, b):
    mean_y = mean_x @ w.T + b
    wm = w @ mom_x
    e2 = jnp.sum(wm * w, axis=1) + 2.0 * b * (w @ mean_x) + b * b
    return mean_y, e2 - mean_y * mean_y


def kernel(xyz1, xyz2, points1, points2, params):
    B = xyz1.shape[0]
    N = xyz1.shape[2]
    K = NSAMPLE
    M_per_g = N * K
    n_mb = M_per_g // _MB
    Mdir = jnp.float32(B * N * K)

    x1 = xyz1.transpose(0, 2, 1)
    x2 = xyz2.transpose(0, 2, 1)
    f1 = points1.transpose(0, 2, 1)
    f2 = points2.transpose(0, 2, 1)

    idx_all = _knn_pallas(x1, x2)

    pad13 = jnp.zeros((2 * B * N, 13), jnp.float32)
    base_coords = jnp.concatenate(
        [x2.reshape(B * N, 3), x1.reshape(B * N, 3)], axis=0)
    base_feats = jnp.concatenate(
        [f2.reshape(B * N, C_IN), f1.reshape(B * N, C_IN)], axis=0)
    t_base = jnp.concatenate(
        [base_coords, jnp.zeros((2 * B * N, 61), jnp.float32), base_feats],
        axis=1)

    offs = (jnp.arange(2 * B, dtype=jnp.int32) * N)[:, None, None]
    idx_flat = (idx_all + offs).reshape(-1)

    gall = _gather_rows(t_base, idx_flat, 512)

    q_pad = jnp.concatenate(
        [jnp.concatenate([x1.reshape(B * N, 3), x2.reshape(B * N, 3)], axis=0),
         pad13], axis=1)
    fq = jnp.concatenate([f1.reshape(B * N, C_IN), f2.reshape(B * N, C_IN)],
                         axis=0)

    wp = params["wn2"]
    Mall = 2 * B * N * K

    wie1_p = jnp.pad(wp["ie1"]["W"].T, ((0, 13), (0, 128 - 64)))
    p1 = pl.pallas_call(
        _p1_body,
        grid=(2 * B, n_mb),
        in_specs=[
            pl.BlockSpec((_MB, 128), lambda g, i: (g * n_mb + i, 0)),
            pl.BlockSpec((_PB, 16), lambda g, i: (g * n_mb + i, 0)),
            pl.BlockSpec((16, 128), lambda g, i: (0, 0)),
            pl.BlockSpec((1, 128), lambda g, i: (0, 0)),
        ],
        out_specs=[
            pl.BlockSpec((_MB, 16), lambda g, i: (g * n_mb + i, 0)),
            pl.BlockSpec((1, 1, 128), lambda g, i: (g, 0, 0)),
            pl.BlockSpec((1, 1, 128), lambda g, i: (g, 0, 0)),
        ],
        out_shape=[
            jax.ShapeDtypeStruct((Mall, 16), jnp.float32),
            jax.ShapeDtypeStruct((2 * B, 1, 128), jnp.float32),
            jax.ShapeDtypeStruct((2 * B, 1, 128), jnp.float32),
        ],
    )(gall, q_pad, wie1_p, _row(wp["ie1"]["b"]))
    pd_all, s0, q0 = p1
    s0 = s0[:, 0, :64].reshape(2, B, 64).sum(1)
    q0 = q0[:, 0, :64].reshape(2, B, 64).sum(1)
    mean0, var0 = _stats_from_sums(s0, q0, Mdir)
    a0, c0 = _bn_ac(mean0, var0, wp["ie_bn"]["g"], wp["ie_bn"]["b"], 1e-05)

    def _cst2(a_list):
        rows = []
        for d in range(2):
            rr = [_row(r[d]) if r.ndim == 2 else _row(r) for r in a_list]
            while len(rr) < 8:
                rr.append(jnp.zeros((1, 128), jnp.float32))
            rows.append(jnp.concatenate(rr, axis=0)[None])
        return jnp.concatenate(rows, axis=0)

    cst_p2 = _cst2([a0, c0])

    d24 = jnp.array([[1.0, 1.0, 10.0, 10.0, 100.0, 100.0, 1000.0, 1000.0] * 3],
                    jnp.float32)
    wie1_t = jnp.pad(wp["ie1"]["W"].T, ((0, 13), (0, 0)))
    wpm1_p = jnp.pad(wp["pm1"]["W"].T, ((0, 0), (0, 64)))
    p2 = pl.pallas_call(
        _p2_body,
        grid=(2 * B, n_mb),
        in_specs=[
            pl.BlockSpec((_MB, 16), lambda g, i: (g * n_mb + i, 0)),
            pl.BlockSpec((1, 8, 128), lambda g, i: (g // 2, 0, 0)),
            pl.BlockSpec((16, 64), lambda g, i: (0, 0)),
            pl.BlockSpec((1, 64), lambda g, i: (0, 0)),
            pl.BlockSpec((64, 64), lambda g, i: (0, 0)),
            pl.BlockSpec((1, 64), lambda g, i: (0, 0)),
            pl.BlockSpec((24, 64), lambda g, i: (0, 0)),
            pl.BlockSpec((1, 64), lambda g, i: (0, 0)),
            pl.BlockSpec((1, 24), lambda g, i: (0, 0)),
            pl.BlockSpec((64, 128), lambda g, i: (0, 0)),
            pl.BlockSpec((1, 128), lambda g, i: (0, 0)),
        ],
        out_specs=[
            pl.BlockSpec((_MB, 64), lambda g, i: (g * n_mb + i, 0)),
            pl.BlockSpec((1, 1, 128), lambda g, i: (g, 0, 0)),
            pl.BlockSpec((1, 1, 128), lambda g, i: (g, 0, 0)),
        ],
        out_shape=[
            jax.ShapeDtypeStruct((Mall, 64), jnp.float32),
            jax.ShapeDtypeStruct((2 * B, 1, 128), jnp.float32),
            jax.ShapeDtypeStruct((2 * B, 1, 128), jnp.float32),
        ],
    )(pd_all, cst_p2, wie1_t, _row(wp["ie1"]["b"], 64),
      wp["ie2"]["W"].T, _row(wp["ie2"]["b"], 64),
      wp["pec"]["W"].T, _row(wp["pec"]["b"], 64), d24,
      wpm1_p, _row(wp["pm1"]["b"]))
    h3_all, s1, q1 = p2
    s1 = s1[:, 0, :64].reshape(2, B, 64).sum(1)
    q1 = q1[:, 0, :64].reshape(2, B, 64).sum(1)
    mean1, var1 = _stats_from_sums(s1, q1, Mdir)
    a1, c1 = _bn_ac(mean1, var1, wp["pm1_bn"]["g"], wp["pm1_bn"]["b"], 0.001)

    cst_p3 = _cst2([a1, c1])
    wpm1_t = wp["pm1"]["W"].T
    bpm1 = _row(wp["pm1"]["b"], 64)

    p3 = pl.pallas_call(
        _p3_body,
        grid=(2 * B, n_mb),
        in_specs=[
            pl.BlockSpec((_MB, 64), lambda g, i: (g * n_mb + i, 0)),
            pl.BlockSpec((1, 8, 128), lambda g, i: (g // 2, 0, 0)),
            pl.BlockSpec((64, 64), lambda g, i: (0, 0)),
            pl.BlockSpec((1, 64), lambda g, i: (0, 0)),
        ],
        out_specs=[
            pl.BlockSpec((1, 1, 64), lambda g, i: (g, 0, 0)),
            pl.BlockSpec((1, 64, 64), lambda g, i: (g, 0, 0)),
        ],
        out_shape=[
            jax.ShapeDtypeStruct((2 * B, 1, 64), jnp.float32),
            jax.ShapeDtypeStruct((2 * B, 64, 64), jnp.float32),
        ],
    )(h3_all, cst_p3, wpm1_t, bpm1)
    s4, m4 = p3
    mean_h4 = s4[:, 0, :].reshape(2, B, 64).sum(1) / Mdir
    mom_h4 = m4.reshape(2, B, 64, 64).sum(1) / Mdir
    mean2 = jnp.stack([_lin_stats(mean_h4[d], mom_h4[d], wp["pm2"]["W"],
                                  wp["pm2"]["b"])[0] for d in range(2)])
    var2 = jnp.stack([_lin_stats(mean_h4[d], mom_h4[d], wp["pm2"]["W"],
                                 wp["pm2"]["b"])[1] for d in range(2)])
    a2, c2 = _bn_ac(mean2, var2, wp["pm2_bn"]["g"], wp["pm2_bn"]["b"], 0.001)

    cst_p4 = _cst2([a1, c1, a2, c2])
    wpm2_t = wp["pm2"]["W"].T
    bpm2 = _row(wp["pm2"]["b"], 64)

    p4 = pl.pallas_call(
        _p4_body,
        grid=(2 * B, n_mb),
        in_specs=[
            pl.BlockSpec((_MB, 64), lambda g, i: (g * n_mb + i, 0)),
            pl.BlockSpec((1, 8, 128), lambda g, i: (g // 2, 0, 0)),
            pl.BlockSpec((64, 64), lambda g, i: (0, 0)),
            pl.BlockSpec((1, 64), lambda g, i: (0, 0)),
            pl.BlockSpec((64, 64), lambda g, i: (0, 0)),
            pl.BlockSpec((1, 64), lambda g, i: (0, 0)),
        ],
        out_specs=[
            pl.BlockSpec((1, 1, 64), lambda g, i: (g, 0, 0)),
            pl.BlockSpec((1, 64, 64), lambda g, i: (g, 0, 0)),
        ],
        out_shape=[
            jax.ShapeDtypeStruct((2 * B, 1, 64), jnp.float32),
            jax.ShapeDtypeStruct((2 * B, 64, 64), jnp.float32),
        ],
    )(h3_all, cst_p4, wpm1_t, bpm1, wpm2_t, bpm2)
    s5, m5 = p4
    mean_h5 = s5[:, 0, :].reshape(2, B, 64).sum(1) / Mdir
    mom_h5 = m5.reshape(2, B, 64, 64).sum(1) / Mdir

    a3l, c3l, w4r, sc_rows = [], [], [], []
    for d in range(2):
        m3, v3 = _lin_stats(mean_h5[d], mom_h5[d], wp["pm3"]["W"], wp["pm3"]["b"])
        a3, c3 = _bn_ac(m3, v3, wp["pm3_bn"]["g"], wp["pm3_bn"]["b"], 0.001)
        w4row = wp["pm4"]["W"][0]
        w_eff = wp["pm3"]["W"].T @ (a3 * w4row)
        b_eff = (wp["pm3"]["b"] * (a3 * w4row)).sum() + (c3 * w4row).sum() \
            + wp["pm4"]["b"][0]
        mean4l = mean_h5[d] @ w_eff + b_eff
        e4 = w_eff @ mom_h5[d] @ w_eff + 2.0 * b_eff * (mean_h5[d] @ w_eff) \
            + b_eff * b_eff
        var4l = e4 - mean4l * mean4l
        a4 = wp["pm4_bn"]["g"][0] / jnp.sqrt(var4l + 0.001)
        c4bn = wp["pm4_bn"]["b"][0] - mean4l * a4
        a3l.append(a3)
        c3l.append(c3)
        w4r.append(w4row)
        sc_rows.append(jnp.stack([wp["pm4"]["b"][0], a4, c4bn]))

    def _cst_dir(d):
        rows = [_row(a1[d]), _row(c1[d]), _row(a2[d]), _row(c2[d]),
                _row(a3l[d]), _row(c3l[d]), _row(w4r[d]), _row(sc_rows[d])]
        return jnp.concatenate(rows, axis=0)[None]

    cst5 = jnp.concatenate([_cst_dir(0), _cst_dir(1)], axis=0)

    wpm3_t = wp["pm3"]["W"].T
    bpm3 = _row(wp["pm3"]["b"], 32)

    def _mlp_w(ml):
        w1, w2, w3 = ml
        return (w1["W"][:, :64].T, w1["W"][:, 64:].T, _row(w1["b"], 64),
                w2["W"].T, _row(w2["b"], 64), w3["W"].T, _row(w3["b"], 128))

    nb2 = B * M_per_g // _MB

    def attn_specs(off):
        return [
            pl.BlockSpec((_MB, 64), lambda i, o=off: (o + i, 0)),
            pl.BlockSpec((1, 8, 128), lambda i: (0, 0, 0)),
            pl.BlockSpec((64, 64), lambda i: (0, 0)),
            pl.BlockSpec((1, 64), lambda i: (0, 0)),
            pl.BlockSpec((64, 64), lambda i: (0, 0)),
            pl.BlockSpec((1, 64), lambda i: (0, 0)),
            pl.BlockSpec((64, 32), lambda i: (0, 0)),
            pl.BlockSpec((1, 32), lambda i: (0, 0)),
            pl.BlockSpec((_PB, 64), lambda i, o=off: (o + i, 0)),
            pl.BlockSpec((_MB, 128), lambda i, o=off: (o + i, 0)),
            pl.BlockSpec((64, 64), lambda i: (0, 0)),
            pl.BlockSpec((64, 64), lambda i: (0, 0)),
            pl.BlockSpec((1, 64), lambda i: (0, 0)),
            pl.BlockSpec((64, 64), lambda i: (0, 0)),
            pl.BlockSpec((1, 64), lambda i: (0, 0)),
            pl.BlockSpec((64, 128), lambda i: (0, 0)),
            pl.BlockSpec((1, 128), lambda i: (0, 0)),
        ]

    cost2 = pl.pallas_call(
        _p5a_body,
        grid=(nb2,),
        in_specs=attn_specs(nb2),
        out_specs=pl.BlockSpec((_PB, 128), lambda i: (i, 0)),
        out_shape=jax.ShapeDtypeStruct((B * N, 128), jnp.float32),
    )(h3_all, cst5[1:2], wpm1_t, bpm1, wpm2_t, bpm2, wpm3_t, bpm3,
      fq, gall, *_mlp_w(params["mlp2"]))

    idx12_flat = idx_flat[:B * N * K]
    gc2 = _gather_rows(cost2, idx12_flat, 512)

    w3 = params["mlp3"]
    w4 = params["mlp4"]
    w3b_p = jnp.pad(w3[0]["W"][:, 128:].T, ((0, 13), (0, 0)))
    c4_rows, flow_pad = pl.pallas_call(
        _p5b_body,
        grid=(nb2,),
        in_specs=attn_specs(0) + [
            pl.BlockSpec((_MB, 16), lambda i: (i, 0)),
            pl.BlockSpec((_MB, 128), lambda i: (i, 0)),
            pl.BlockSpec((128, 64), lambda i: (0, 0)),
            pl.BlockSpec((16, 64), lambda i: (0, 0)),
            pl.BlockSpec((1, 64), lambda i: (0, 0)),
            pl.BlockSpec((64, 64), lambda i: (0, 0)),
            pl.BlockSpec((1, 64), lambda i: (0, 0)),
            pl.BlockSpec((64, 128), lambda i: (0, 0)),
            pl.BlockSpec((1, 128), lambda i: (0, 0)),
            pl.BlockSpec((128, 64), lambda i: (0, 0)),
            pl.BlockSpec((128, 64), lambda i: (0, 0)),
            pl.BlockSpec((1, 64), lambda i: (0, 0)),
            pl.BlockSpec((64, 64), lambda i: (0, 0)),
            pl.BlockSpec((1, 64), lambda i: (0, 0)),
            pl.BlockSpec((64, 128), lambda i: (0, 0)),
            pl.BlockSpec((1, 128), lambda i: (0, 0)),
        ],
        out_specs=[
            pl.BlockSpec((_PB, 128), lambda i: (i, 0)),
            pl.BlockSpec((_PB, 16), lambda i: (i, 0)),
        ],
        out_shape=[
            jax.ShapeDtypeStruct((B * N, 128), jnp.float32),
            jax.ShapeDtypeStruct((B * N, 16), jnp.float32),
        ],
    )(h3_all, cst5[0:1], wpm1_t, bpm1, wpm2_t, bpm2, wpm3_t, bpm3,
      fq, gall, *_mlp_w(params["mlp1"]),
      pd_all, gc2,
      w3[0]["W"][:, :128].T, w3b_p, _row(w3[0]["b"], 64),
      w3[1]["W"].T, _row(w3[1]["b"], 64), w3[2]["W"].T, _row(w3[2]["b"], 128),
      w4[0]["W"][:, :128].T, w4[0]["W"][:, 128:].T, _row(w4[0]["b"], 64),
      w4[1]["W"].T, _row(w4[1]["b"], 64), w4[2]["W"].T, _row(w4[2]["b"], 128))

    c4 = c4_rows.reshape(B, N, 128).transpose(0, 2, 1)
    flow = flow_pad[:, :3].reshape(B, N, 3).transpose(0, 2, 1)
    return c4, flow


# speedup vs baseline: 3.5710x; 1.0099x over previous
import functools
import math

import jax
import jax.numpy as jnp
from jax import lax
from jax.experimental import pallas as pl
from jax.experimental.pallas import tpu as pltpu
from jax.experimental.pallas import tpu_sc as plsc

C_IN = 64
NSAMPLE = 16
VOXEL = 0.25

_QB = 512
_MB = 4096
_PB = _MB // NSAMPLE

_HIGH = lax.Precision.HIGHEST


def _mm(x, wt, precision=None):
    return lax.dot_general(x, wt, (((1,), (0,)), ((), ())),
                           precision=precision,
                           preferred_element_type=jnp.float32)


def _leaky(x):
    return jnp.where(x > 0, x, 0.1 * x)


def _knn_body(qmat_ref, bmat_ref, out_ref):
    q = qmat_ref[0]
    bm = bmat_ref[0]
    s = jnp.dot(q, bm, preferred_element_type=jnp.float32)
    qq = (q[:, 0:1] * q[:, 0:1] + q[:, 1:2] * q[:, 1:2]) + q[:, 2:3] * q[:, 2:3]
    bb = bm[3:4, :]
    d = (qq - 2.0 * s) + bb
    iota = lax.broadcasted_iota(jnp.int32, d.shape, 1)
    big = jnp.int32(2 ** 30)
    for r in range(NSAMPLE):
        m = jnp.min(d, axis=1, keepdims=True)
        sel = jnp.min(jnp.where(d <= m, iota, big), axis=1, keepdims=True)
        out_ref[0, :, r:r + 1] = sel
        d = jnp.where(iota == sel, jnp.float32(jnp.inf), d)


def _knn_pallas(x1, x2):
    B, N, _ = x1.shape
    q_all = jnp.concatenate([x1, x2], axis=0)
    b_all = jnp.concatenate([x2, x1], axis=0)
    zeros1 = jnp.zeros((2 * B, N, 1), jnp.float32)
    zeros4 = jnp.zeros((2 * B, N, 4), jnp.float32)
    qmat = jnp.concatenate([q_all, zeros1, zeros4], axis=-1)
    bb = jnp.sum(b_all * b_all, axis=-1, keepdims=True)
    bmat = jnp.concatenate([b_all, bb, zeros4], axis=-1).transpose(0, 2, 1)

    return pl.pallas_call(
        _knn_body,
        grid=(2 * B, N // _QB),
        in_specs=[
            pl.BlockSpec((1, _QB, 8), lambda g, i: (g, i, 0)),
            pl.BlockSpec((1, 8, N), lambda g, i: (g, 0, 0)),
        ],
        out_specs=pl.BlockSpec((1, _QB, NSAMPLE), lambda g, i: (g, i, 0)),
        out_shape=jax.ShapeDtypeStruct((2 * B, N, NSAMPLE), jnp.int32),
    )(qmat, bmat)


def _gather_rows(table, idx, chunk):
    T, D = table.shape
    Mtot = idx.shape[0]
    NC, NS = 2, 16
    NW = NC * NS
    per_w = Mtot // NW
    n_iter = per_w // chunk
    mesh = plsc.VectorSubcoreMesh(core_axis_name="c", subcore_axis_name="s")

    @functools.partial(
        pl.kernel, mesh=mesh,
        out_type=jax.ShapeDtypeStruct((Mtot, D), jnp.float32),
        scratch_types=[
            pltpu.VMEM((chunk,), jnp.int32),
            pltpu.VMEM((chunk, D), jnp.float32),
            pltpu.SemaphoreType.DMA,
        ],
    )
    def k(table_hbm, idx_hbm, out_hbm, idx_v, rows_v, sem):
        wid = lax.axis_index("s") * NC + lax.axis_index("c")
        base = wid * per_w

        def body(j, carry):
            off = base + j * chunk
            pltpu.sync_copy(idx_hbm.at[pl.ds(off, chunk)], idx_v)
            pltpu.async_copy(table_hbm.at[idx_v], rows_v, sem).wait()
            pltpu.sync_copy(rows_v, out_hbm.at[pl.ds(off, chunk)])
            return carry

        lax.fori_loop(0, n_iter, body, 0)

    return k(table, idx)


def _p1_body(nx_ref, xq_ref, w_ref, b_ref, pd_ref, s_ref, q_ref):
    i = pl.program_id(1)
    xq = xq_ref[...]
    xq_rep = jnp.broadcast_to(xq[:, None, :], (_PB, NSAMPLE, 16)).reshape(_MB, 16)
    pd = nx_ref[:, 0:16] - xq_rep
    pd_ref[...] = pd
    h0 = _mm(pd, w_ref[...]) + b_ref[...]

    @pl.when(i == 0)
    def _():
        s_ref[...] = jnp.zeros_like(s_ref)
        q_ref[...] = jnp.zeros_like(q_ref)

    s_ref[...] += jnp.sum(h0, axis=0, keepdims=True)[None]
    q_ref[...] += jnp.sum(h0 * h0, axis=0, keepdims=True)[None]


def _p2_body(pd_ref, cst_ref, wie1_ref, bie1_ref, wie2_ref, bie2_ref,
             pec_ref, bpec_ref, d24_ref, wpm1_ref, bpm1_ref,
             h3_ref, s_ref, q_ref):
    i = pl.program_id(1)
    pd = pd_ref[...]
    h0 = _mm(pd, wie1_ref[...]) + bie1_ref[...]
    a0 = cst_ref[0, 0:1, 0:64]
    c0 = cst_ref[0, 1:2, 0:64]
    h1 = jax.nn.relu(h0 * a0 + c0)
    h2 = _mm(h1, wie2_ref[...]) + bie2_ref[...]

    r = VOXEL
    dis_voxel = jnp.round(pd / r)
    pe_raw = (pd - dis_voxel * r) / r
    e = pe_raw / (1.0 + 1e-06) * (2.0 * math.pi)
    cols = [jnp.broadcast_to(e[:, c:c + 1], (_MB, 8)) for c in range(3)]
    e24 = jnp.concatenate(cols, axis=1)
    q24 = e24 / d24_ref[...]
    lane = lax.broadcasted_iota(jnp.int32, (_MB, 24), 1)
    feats = jnp.where(lane % 2 == 0, jnp.sin(q24), jnp.cos(q24))
    pos = _mm(feats, pec_ref[...]) + bpec_ref[...]

    h3 = h2 + pos
    h3_ref[...] = h3
    g1 = _mm(h3, wpm1_ref[...]) + bpm1_ref[...]

    @pl.when(i == 0)
    def _():
        s_ref[...] = jnp.zeros_like(s_ref)
        q_ref[...] = jnp.zeros_like(q_ref)

    s_ref[...] += jnp.sum(g1, axis=0, keepdims=True)[None]
    q_ref[...] += jnp.sum(g1 * g1, axis=0, keepdims=True)[None]


def _h4_from_h3(h3, cst_ref, wpm1_ref, bpm1_ref):
    g1 = _mm(h3, wpm1_ref[...]) + bpm1_ref[...]
    a1 = cst_ref[0, 0:1, 0:64]
    c1 = cst_ref[0, 1:2, 0:64]
    return jax.nn.relu(g1 * a1 + c1)


def _p3_body(h3_ref, cst_ref, wpm1_ref, bpm1_ref, s_ref, m_ref):
    i = pl.program_id(1)
    h4 = _h4_from_h3(h3_ref[...], cst_ref, wpm1_ref, bpm1_ref)

    @pl.when(i == 0)
    def _():
        s_ref[...] = jnp.zeros_like(s_ref)
        m_ref[...] = jnp.zeros_like(m_ref)

    s_ref[...] += jnp.sum(h4, axis=0, keepdims=True)[None]
    mom = lax.dot_general(h4, h4, (((0,), (0,)), ((), ())),
                          precision=_HIGH, preferred_element_type=jnp.float32)
    m_ref[...] += mom[None]


def _h5_from_h3(h3, cst_ref, wpm1_ref, bpm1_ref, wpm2_ref, bpm2_ref):
    h4 = _h4_from_h3(h3, cst_ref, wpm1_ref, bpm1_ref)
    g2 = _mm(h4, wpm2_ref[...]) + bpm2_ref[...]
    a2 = cst_ref[0, 2:3, 0:64]
    c2 = cst_ref[0, 3:4, 0:64]
    return jax.nn.relu(g2 * a2 + c2)


def _p4_body(h3_ref, cst_ref, wpm1_ref, bpm1_ref, wpm2_ref, bpm2_ref,
             s_ref, m_ref):
    i = pl.program_id(1)
    h5 = _h5_from_h3(h3_ref[...], cst_ref, wpm1_ref, bpm1_ref, wpm2_ref, bpm2_ref)

    @pl.when(i == 0)
    def _():
        s_ref[...] = jnp.zeros_like(s_ref)
        m_ref[...] = jnp.zeros_like(m_ref)

    s_ref[...] += jnp.sum(h5, axis=0, keepdims=True)[None]
    mom = lax.dot_general(h5, h5, (((0,), (0,)), ((), ())),
                          precision=_HIGH, preferred_element_type=jnp.float32)
    m_ref[...] += mom[None]


def _attn_from_h3(h3, cst_ref, wpm1_ref, bpm1_ref, wpm2_ref, bpm2_ref,
                  wpm3_ref, bpm3_ref):
    h5 = _h5_from_h3(h3, cst_ref, wpm1_ref, bpm1_ref, wpm2_ref, bpm2_ref)
    g3 = _mm(h5, wpm3_ref[...]) + bpm3_ref[...]
    a3 = cst_ref[0, 4:5, 0:32]
    c3 = cst_ref[0, 5:6, 0:32]
    h6 = g3 * a3 + c3
    w4 = cst_ref[0, 6:7, 0:32]
    b4 = cst_ref[0, 7:8, 0:1]
    a4 = cst_ref[0, 7:8, 1:2]
    c4 = cst_ref[0, 7:8, 2:3]
    g4 = jnp.sum(h6 * w4, axis=1, keepdims=True) + b4
    h7 = g4 * a4 + c4
    att = h7.reshape(_PB, NSAMPLE)
    mx = jnp.max(att, axis=1, keepdims=True)
    ex = jnp.exp(att - mx)
    return ex / jnp.sum(ex, axis=1, keepdims=True)


def _feat_mlp(fq, gfb, wq_ref, wg_ref, b1_ref, w2_ref, b2_ref, w3_ref, b3_ref):
    uq = _mm(fq, wq_ref[...])
    urep = jnp.broadcast_to(uq[:, None, :], (_PB, NSAMPLE, 64)).reshape(_MB, 64)
    t1 = _leaky(urep + _mm(gfb[:, 64:128], wg_ref[...]) + b1_ref[...])
    t2 = _leaky(_mm(t1, w2_ref[...]) + b2_ref[...])
    return _leaky(_mm(t2, w3_ref[...]) + b3_ref[...])


def _wsum(att, x, d):
    return jnp.sum(att[:, :, None] * x.reshape(_PB, NSAMPLE, d), axis=1)


def _p5a_body(h3_ref, cst_ref, wpm1_ref, bpm1_ref, wpm2_ref, bpm2_ref,
              wpm3_ref, bpm3_ref, fq_ref, gfb_ref,
              wq_ref, wg_ref, b1_ref, w2_ref, b2_ref, w3_ref, b3_ref,
              cost_ref):
    att = _attn_from_h3(h3_ref[...], cst_ref, wpm1_ref, bpm1_ref,
                        wpm2_ref, bpm2_ref, wpm3_ref, bpm3_ref)
    c = _feat_mlp(fq_ref[...], gfb_ref[...], wq_ref, wg_ref, b1_ref,
                  w2_ref, b2_ref, w3_ref, b3_ref)
    cost_ref[...] = _wsum(att, c, 128)


def _p5b_body(h3_ref, cst_ref, wpm1_ref, bpm1_ref, wpm2_ref, bpm2_ref,
              wpm3_ref, bpm3_ref, fq_ref, gfb_ref,
              wq_ref, wg_ref, b1_ref, w2_ref, b2_ref, w3_ref, b3_ref,
              pd_ref, gc2_ref, w3a_ref, w3b_ref, b31_ref, w32_ref, b32_ref,
              w33_ref, b33_ref,
              w4a_ref, w4b_ref, b41_ref, w42_ref, b42_ref, w43_ref, b43_ref,
              c4_ref, flow_ref):
    att = _attn_from_h3(h3_ref[...], cst_ref, wpm1_ref, bpm1_ref,
                        wpm2_ref, bpm2_ref, wpm3_ref, bpm3_ref)
    c1 = _feat_mlp(fq_ref[...], gfb_ref[...], wq_ref, wg_ref, b1_ref,
                   w2_ref, b2_ref, w3_ref, b3_ref)
    cost1 = _wsum(att, c1, 128)

    pd = pd_ref[...]
    t = _leaky(_mm(gc2_ref[...], w3a_ref[...]) + _mm(pd, w3b_ref[...]) + b31_ref[...])
    t2 = _leaky(_mm(t, w32_ref[...]) + b32_ref[...])
    c3 = _leaky(_mm(t2, w33_ref[...]) + b33_ref[...])
    cost21 = _wsum(att, c3, 128)

    flow_ref[...] = _wsum(att, pd, 16)

    m1 = _leaky(_mm(cost1, w4a_ref[...]) + _mm(cost21, w4b_ref[...]) + b41_ref[...])
    m2 = _leaky(_mm(m1, w42_ref[...]) + b42_ref[...])
    c4_ref[...] = _leaky(_mm(m2, w43_ref[...]) + b43_ref[...])


def _row(v, width=128):
    v = jnp.asarray(v, jnp.float32).reshape(1, -1)
    return jnp.pad(v, ((0, 0), (0, width - v.shape[1])))


def _bn_ac(mean, var, g, b, eps):
    a = g / jnp.sqrt(var + eps)
    return a, b - mean * a


def _stats_from_sums(s, q, count):
    mean = s / count
    var = q / count - mean * mean
    return mean, var


def _lin_stats(mean_x, mom_x, w, b):
    mean_y = mean_x @ w.T + b
    wm = w @ mom_x
    e2 = jnp.sum(wm * w, axis=1) + 2.0 * b * (w @ mean_x) + b * b
    return mean_y, e2 - mean_y * mean_y


def kernel(xyz1, xyz2, points1, points2, params):
    B = xyz1.shape[0]
    N = xyz1.shape[2]
    K = NSAMPLE
    M_per_g = N * K
    n_mb = M_per_g // _MB
    Mdir = jnp.float32(B * N * K)

    x1 = xyz1.transpose(0, 2, 1)
    x2 = xyz2.transpose(0, 2, 1)
    f1 = points1.transpose(0, 2, 1)
    f2 = points2.transpose(0, 2, 1)

    idx_all = _knn_pallas(x1, x2)

    pad13 = jnp.zeros((2 * B * N, 13), jnp.float32)
    base_coords = jnp.concatenate(
        [x2.reshape(B * N, 3), x1.reshape(B * N, 3)], axis=0)
    base_feats = jnp.concatenate(
        [f2.reshape(B * N, C_IN), f1.reshape(B * N, C_IN)], axis=0)
    t_base = jnp.concatenate(
        [base_coords, jnp.zeros((2 * B * N, 61), jnp.float32), base_feats],
        axis=1)

    offs = (jnp.arange(2 * B, dtype=jnp.int32) * N)[:, None, None]
    idx_flat = (idx_all + offs).reshape(-1)

    gall = _gather_rows(t_base, idx_flat, 512)

    q_pad = jnp.concatenate(
        [jnp.concatenate([x1.reshape(B * N, 3), x2.reshape(B * N, 3)], axis=0),
         pad13], axis=1)
    fq = jnp.concatenate([f1.reshape(B * N, C_IN), f2.reshape(B * N, C_IN)],
                         axis=0)

    wp = params["wn2"]
    Mall = 2 * B * N * K

    wie1_p = jnp.pad(wp["ie1"]["W"].T, ((0, 13), (0, 128 - 64)))
    p1 = pl.pallas_call(
        _p1_body,
        grid=(2 * B, n_mb),
        in_specs=[
            pl.BlockSpec((_MB, 128), lambda g, i: (g * n_mb + i, 0)),
            pl.BlockSpec((_PB, 16), lambda g, i: (g * n_mb + i, 0)),
            pl.BlockSpec((16, 128), lambda g, i: (0, 0)),
            pl.BlockSpec((1, 128), lambda g, i: (0, 0)),
        ],
        out_specs=[
            pl.BlockSpec((_MB, 16), lambda g, i: (g * n_mb + i, 0)),
            pl.BlockSpec((1, 1, 128), lambda g, i: (g, 0, 0)),
            pl.BlockSpec((1, 1, 128), lambda g, i: (g, 0, 0)),
        ],
        out_shape=[
            jax.ShapeDtypeStruct((Mall, 16), jnp.float32),
            jax.ShapeDtypeStruct((2 * B, 1, 128), jnp.float32),
            jax.ShapeDtypeStruct((2 * B, 1, 128), jnp.float32),
        ],
    )(gall, q_pad, wie1_p, _row(wp["ie1"]["b"]))
    pd_all, s0, q0 = p1
    s0 = s0[:, 0, :64].reshape(2, B, 64).sum(1)
    q0 = q0[:, 0, :64].reshape(2, B, 64).sum(1)
    mean0, var0 = _stats_from_sums(s0, q0, Mdir)
    a0, c0 = _bn_ac(mean0, var0, wp["ie_bn"]["g"], wp["ie_bn"]["b"], 1e-05)

    def _cst2(a_list):
        rows = []
        for d in range(2):
            rr = [_row(r[d]) if r.ndim == 2 else _row(r) for r in a_list]
            while len(rr) < 8:
                rr.append(jnp.zeros((1, 128), jnp.float32))
            rows.append(jnp.concatenate(rr, axis=0)[None])
        return jnp.concatenate(rows, axis=0)

    cst_p2 = _cst2([a0, c0])

    d24 = jnp.array([[1.0, 1.0, 10.0, 10.0, 100.0, 100.0, 1000.0, 1000.0] * 3],
                    jnp.float32)
    wie1_t = jnp.pad(wp["ie1"]["W"].T, ((0, 13), (0, 0)))
    wpm1_p = jnp.pad(wp["pm1"]["W"].T, ((0, 0), (0, 64)))
    p2 = pl.pallas_call(
        _p2_body,
        grid=(2 * B, n_mb),
        in_specs=[
            pl.BlockSpec((_MB, 16), lambda g, i: (g * n_mb + i, 0)),
            pl.BlockSpec((1, 8, 128), lambda g, i: (g // 2, 0, 0)),
            pl.BlockSpec((16, 64), lambda g, i: (0, 0)),
            pl.BlockSpec((1, 64), lambda g, i: (0, 0)),
            pl.BlockSpec((64, 64), lambda g, i: (0, 0)),
            pl.BlockSpec((1, 64), lambda g, i: (0, 0)),
            pl.BlockSpec((24, 64), lambda g, i: (0, 0)),
            pl.BlockSpec((1, 64), lambda g, i: (0, 0)),
            pl.BlockSpec((1, 24), lambda g, i: (0, 0)),
            pl.BlockSpec((64, 128), lambda g, i: (0, 0)),
            pl.BlockSpec((1, 128), lambda g, i: (0, 0)),
        ],
        out_specs=[
            pl.BlockSpec((_MB, 64), lambda g, i: (g * n_mb + i, 0)),
            pl.BlockSpec((1, 1, 128), lambda g, i: (g, 0, 0)),
            pl.BlockSpec((1, 1, 128), lambda g, i: (g, 0, 0)),
        ],
        out_shape=[
            jax.ShapeDtypeStruct((Mall, 64), jnp.float32),
            jax.ShapeDtypeStruct((2 * B, 1, 128), jnp.float32),
            jax.ShapeDtypeStruct((2 * B, 1, 128), jnp.float32),
        ],
    )(pd_all, cst_p2, wie1_t, _row(wp["ie1"]["b"], 64),
      wp["ie2"]["W"].T, _row(wp["ie2"]["b"], 64),
      wp["pec"]["W"].T, _row(wp["pec"]["b"], 64), d24,
      wpm1_p, _row(wp["pm1"]["b"]))
    h3_all, s1, q1 = p2
    s1 = s1[:, 0, :64].reshape(2, B, 64).sum(1)
    q1 = q1[:, 0, :64].reshape(2, B, 64).sum(1)
    mean1, var1 = _stats_from_sums(s1, q1, Mdir)
    a1, c1 = _bn_ac(mean1, var1, wp["pm1_bn"]["g"], wp["pm1_bn"]["b"], 0.001)

    cst_p3 = _cst2([a1, c1])
    wpm1_t = wp["pm1"]["W"].T
    bpm1 = _row(wp["pm1"]["b"], 64)

    p3 = pl.pallas_call(
        _p3_body,
        grid=(2 * B, n_mb),
        in_specs=[
            pl.BlockSpec((_MB, 64), lambda g, i: (g * n_mb + i, 0)),
            pl.BlockSpec((1, 8, 128), lambda g, i: (g // 2, 0, 0)),
            pl.BlockSpec((64, 64), lambda g, i: (0, 0)),
            pl.BlockSpec((1, 64), lambda g, i: (0, 0)),
        ],
        out_specs=[
            pl.BlockSpec((1, 1, 64), lambda g, i: (g, 0, 0)),
            pl.BlockSpec((1, 64, 64), lambda g, i: (g, 0, 0)),
        ],
        out_shape=[
            jax.ShapeDtypeStruct((2 * B, 1, 64), jnp.float32),
            jax.ShapeDtypeStruct((2 * B, 64, 64), jnp.float32),
        ],
    )(h3_all, cst_p3, wpm1_t, bpm1)
    s4, m4 = p3
    mean_h4 = s4[:, 0, :].reshape(2, B, 64).sum(1) / Mdir
    mom_h4 = m4.reshape(2, B, 64, 64).sum(1) / Mdir
    mean2 = jnp.stack([_lin_stats(mean_h4[d], mom_h4[d], wp["pm2"]["W"],
                                  wp["pm2"]["b"])[0] for d in range(2)])
    var2 = jnp.stack([_lin_stats(mean_h4[d], mom_h4[d], wp["pm2"]["W"],
                                 wp["pm2"]["b"])[1] for d in range(2)])
    a2, c2 = _bn_ac(mean2, var2, wp["pm2_bn"]["g"], wp["pm2_bn"]["b"], 0.001)

    cst_p4 = _cst2([a1, c1, a2, c2])
    wpm2_t = wp["pm2"]["W"].T
    bpm2 = _row(wp["pm2"]["b"], 64)

    p4 = pl.pallas_call(
        _p4_body,
        grid=(2 * B, n_mb),
        in_specs=[
            pl.BlockSpec((_MB, 64), lambda g, i: (g * n_mb + i, 0)),
            pl.BlockSpec((1, 8, 128), lambda g, i: (g // 2, 0, 0)),
            pl.BlockSpec((64, 64), lambda g, i: (0, 0)),
            pl.BlockSpec((1, 64), lambda g, i: (0, 0)),
            pl.BlockSpec((64, 64), lambda g, i: (0, 0)),
            pl.BlockSpec((1, 64), lambda g, i: (0, 0)),
        ],
        out_specs=[
            pl.BlockSpec((1, 1, 64), lambda g, i: (g, 0, 0)),
            pl.BlockSpec((1, 64, 64), lambda g, i: (g, 0, 0)),
        ],
        out_shape=[
            jax.ShapeDtypeStruct((2 * B, 1, 64), jnp.float32),
            jax.ShapeDtypeStruct((2 * B, 64, 64), jnp.float32),
        ],
    )(h3_all, cst_p4, wpm1_t, bpm1, wpm2_t, bpm2)
    s5, m5 = p4
    mean_h5 = s5[:, 0, :].reshape(2, B, 64).sum(1) / Mdir
    mom_h5 = m5.reshape(2, B, 64, 64).sum(1) / Mdir

    a3l, c3l, w4r, sc_rows = [], [], [], []
    for d in range(2):
        m3, v3 = _lin_stats(mean_h5[d], mom_h5[d], wp["pm3"]["W"], wp["pm3"]["b"])
        a3, c3 = _bn_ac(m3, v3, wp["pm3_bn"]["g"], wp["pm3_bn"]["b"], 0.001)
        w4row = wp["pm4"]["W"][0]
        w_eff = wp["pm3"]["W"].T @ (a3 * w4row)
        b_eff = (wp["pm3"]["b"] * (a3 * w4row)).sum() + (c3 * w4row).sum() \
            + wp["pm4"]["b"][0]
        mean4l = mean_h5[d] @ w_eff + b_eff
        e4 = w_eff @ mom_h5[d] @ w_eff + 2.0 * b_eff * (mean_h5[d] @ w_eff) \
            + b_eff * b_eff
        var4l = e4 - mean4l * mean4l
        a4 = wp["pm4_bn"]["g"][0] / jnp.sqrt(var4l + 0.001)
        c4bn = wp["pm4_bn"]["b"][0] - mean4l * a4
        a3l.append(a3)
        c3l.append(c3)
        w4r.append(w4row)
        sc_rows.append(jnp.stack([wp["pm4"]["b"][0], a4, c4bn]))

    def _cst_dir(d):
        rows = [_row(a1[d]), _row(c1[d]), _row(a2[d]), _row(c2[d]),
                _row(a3l[d]), _row(c3l[d]), _row(w4r[d]), _row(sc_rows[d])]
        return jnp.concatenate(rows, axis=0)[None]

    cst5 = jnp.concatenate([_cst_dir(0), _cst_dir(1)], axis=0)

    wpm3_t = wp["pm3"]["W"].T
    bpm3 = _row(wp["pm3"]["b"], 32)

    def _mlp_w(ml):
        w1, w2, w3 = ml
        return (w1["W"][:, :64].T, w1["W"][:, 64:].T, _row(w1["b"], 64),
                w2["W"].T, _row(w2["b"], 64), w3["W"].T, _row(w3["b"], 128))

    nb2 = B * M_per_g // _MB

    def attn_specs(off):
        return [
            pl.BlockSpec((_MB, 64), lambda i, o=off: (o + i, 0)),
            pl.BlockSpec((1, 8, 128), lambda i: (0, 0, 0)),
            pl.BlockSpec((64, 64), lambda i: (0, 0)),
            pl.BlockSpec((1, 64), lambda i: (0, 0)),
            pl.BlockSpec((64, 64), lambda i: (0, 0)),
            pl.BlockSpec((1, 64), lambda i: (0, 0)),
            pl.BlockSpec((64, 32), lambda i: (0, 0)),
            pl.BlockSpec((1, 32), lambda i: (0, 0)),
            pl.BlockSpec((_PB, 64), lambda i, o=off: (o + i, 0)),
            pl.BlockSpec((_MB, 128), lambda i, o=off: (o + i, 0)),
            pl.BlockSpec((64, 64), lambda i: (0, 0)),
            pl.BlockSpec((64, 64), lambda i: (0, 0)),
            pl.BlockSpec((1, 64), lambda i: (0, 0)),
            pl.BlockSpec((64, 64), lambda i: (0, 0)),
            pl.BlockSpec((1, 64), lambda i: (0, 0)),
            pl.BlockSpec((64, 128), lambda i: (0, 0)),
            pl.BlockSpec((1, 128), lambda i: (0, 0)),
        ]

    cost2 = pl.pallas_call(
        _p5a_body,
        grid=(nb2,),
        in_specs=attn_specs(nb2),
        out_specs=pl.BlockSpec((_PB, 128), lambda i: (i, 0)),
        out_shape=jax.ShapeDtypeStruct((B * N, 128), jnp.float32),
    )(h3_all, cst5[1:2], wpm1_t, bpm1, wpm2_t, bpm2, wpm3_t, bpm3,
      fq, gall, *_mlp_w(params["mlp2"]))

    idx12_flat = idx_flat[:B * N * K]
    gc2 = _gather_rows(cost2, idx12_flat, 512)

    w3 = params["mlp3"]
    w4 = params["mlp4"]
    w3b_p = jnp.pad(w3[0]["W"][:, 128:].T, ((0, 13), (0, 0)))
    c4_rows, flow_pad = pl.pallas_call(
        _p5b_body,
        grid=(nb2,),
        in_specs=attn_specs(0) + [
            pl.BlockSpec((_MB, 16), lambda i: (i, 0)),
            pl.BlockSpec((_MB, 128), lambda i: (i, 0)),
            pl.BlockSpec((128, 64), lambda i: (0, 0)),
            pl.BlockSpec((16, 64), lambda i: (0, 0)),
            pl.BlockSpec((1, 64), lambda i: (0, 0)),
            pl.BlockSpec((64, 64), lambda i: (0, 0)),
            pl.BlockSpec((1, 64), lambda i: (0, 0)),
            pl.BlockSpec((64, 128), lambda i: (0, 0)),
            pl.BlockSpec((1, 128), lambda i: (0, 0)),
            pl.BlockSpec((128, 64), lambda i: (0, 0)),
            pl.BlockSpec((128, 64), lambda i: (0, 0)),
            pl.BlockSpec((1, 64), lambda i: (0, 0)),
            pl.BlockSpec((64, 64), lambda i: (0, 0)),
            pl.BlockSpec((1, 64), lambda i: (0, 0)),
            pl.BlockSpec((64, 128), lambda i: (0, 0)),
            pl.BlockSpec((1, 128), lambda i: (0, 0)),
        ],
        out_specs=[
            pl.BlockSpec((_PB, 128), lambda i: (i, 0)),
            pl.BlockSpec((_PB, 16), lambda i: (i, 0)),
        ],
        out_shape=[
            jax.ShapeDtypeStruct((B * N, 128), jnp.float32),
            jax.ShapeDtypeStruct((B * N, 16), jnp.float32),
        ],
    )(h3_all, cst5[0:1], wpm1_t, bpm1, wpm2_t, bpm2, wpm3_t, bpm3,
      fq, gall, *_mlp_w(params["mlp1"]),
      pd_all, gc2,
      w3[0]["W"][:, :128].T, w3b_p, _row(w3[0]["b"], 64),
      w3[1]["W"].T, _row(w3[1]["b"], 64), w3[2]["W"].T, _row(w3[2]["b"], 128),
      w4[0]["W"][:, :128].T, w4[0]["W"][:, 128:].T, _row(w4[0]["b"], 64),
      w4[1]["W"].T, _row(w4[1]["b"], 64), w4[2]["W"].T, _row(w4[2]["b"], 128))

    c4 = c4_rows.reshape(B, N, 128).transpose(0, 2, 1)
    flow = flow_pad[:, :3].reshape(B, N, 3).transpose(0, 2, 1)
    return c4, flow
